# baseline (speedup 1.0000x reference)
import functools

import jax
import jax.numpy as jnp
from jax import lax
from jax.experimental import pallas as pl
from jax.experimental.pallas import tpu as pltpu
from jax.experimental.pallas import tpu_sc as plsc

_NUM_ENTITIES = 1000000
_NUM_RELATIONS = 1000
_EMBED_DIM = 64
_BATCH = 16384

_NC = 2
_NS = 16
_L = 16
_NW = _NC * _NS
_BPW = _BATCH // _NW
_CHUNK = 128
_NCHUNK = _BPW // _CHUNK
_DBLK = _EMBED_DIM // _L
_PW = 2 * _EMBED_DIM

_GDN = lax.GatherDimensionNumbers(
    offset_dims=(), collapsed_slice_dims=(0,), start_index_map=(0,))


def _permute(v, idx):
    return lax.gather(v, idx[:, None], _GDN, slice_sizes=(1,),
                      mode=lax.GatherScatterMode.PROMISE_IN_BOUNDS)


def _hsum_tree(vs, lanes):
    for d in (1, 2, 4, 8):
        perm = lanes ^ d
        mask = (lanes & d) != 0
        nxt = []
        for i in range(0, len(vs), 2):
            a = vs[i] + _permute(vs[i], perm)
            b = vs[i + 1] + _permute(vs[i + 1], perm)
            nxt.append(jnp.where(mask, b, a))
        vs = nxt
    return vs[0]


def _dm_body(e1i, reli, e2i, ent, rel, out,
             idx_e1, idx_r, idx_e2, pid_e1, pid_r, pid_e2,
             buf_e1, buf_r, buf_e2, out_v, sem):
    w = lax.axis_index("s") * _NC + lax.axis_index("c")
    base = w * _BPW
    crow = w * _NCHUNK

    pltpu.sync_copy(e1i.at[pl.ds(crow, _NCHUNK)], idx_e1)
    pltpu.sync_copy(reli.at[pl.ds(crow, _NCHUNK)], idx_r)
    pltpu.sync_copy(e2i.at[pl.ds(crow, _NCHUNK)], idx_e2)

    def halve(g, carry):
        c = g // (_CHUNK // _L)
        o = (g % (_CHUNK // _L)) * _L
        sl = pl.ds(o, _L)
        pid_e1[c, sl] = idx_e1[c, sl] >> 1
        pid_r[c, sl] = idx_r[c, sl] >> 1
        pid_e2[c, sl] = idx_e2[c, sl] >> 1
        return carry

    lax.fori_loop(0, _NCHUNK * (_CHUNK // _L), halve, 0)

    def issue(c):
        b = c % 2
        pltpu.async_copy(ent.at[pid_e1.at[c]], buf_e1.at[b], sem)
        pltpu.async_copy(rel.at[pid_r.at[c]], buf_r.at[b], sem)
        pltpu.async_copy(ent.at[pid_e2.at[c]], buf_e2.at[b], sem)

    def drain(c):
        b = c % 2
        pltpu.make_async_copy(ent.at[pid_e1.at[c]], buf_e1.at[b], sem).wait()
        pltpu.make_async_copy(rel.at[pid_r.at[c]], buf_r.at[b], sem).wait()
        pltpu.make_async_copy(ent.at[pid_e2.at[c]], buf_e2.at[b], sem).wait()

    lanes = lax.iota(jnp.int32, _L)

    def compute_chunk(c):
        b = c % 2

        def blk_body(g, carry):
            gb = g * _L
            p1v = idx_e1[c, pl.ds(gb, _L)] & 1
            prv = idx_r[c, pl.ds(gb, _L)] & 1
            p2v = idx_e2[c, pl.ds(gb, _L)] & 1
            partials = []
            for i in range(_L):
                r = gb + i
                o1 = p1v[i] * _EMBED_DIM
                orr = prv[i] * _EMBED_DIM
                o2 = p2v[i] * _EMBED_DIM
                acc = (buf_e1[b, r, pl.ds(o1, _L)]
                       * buf_r[b, r, pl.ds(orr, _L)]
                       * buf_e2[b, r, pl.ds(o2, _L)])
                for k in range(1, _DBLK):
                    acc = acc + (buf_e1[b, r, pl.ds(o1 + k * _L, _L)]
                                 * buf_r[b, r, pl.ds(orr + k * _L, _L)]
                                 * buf_e2[b, r, pl.ds(o2 + k * _L, _L)])
                partials.append(acc)
            tot = _hsum_tree(partials, lanes)
            y = 1.0 / (1.0 + jnp.exp(-tot))
            out_v[pl.ds(c * _CHUNK + gb, _L)] = y
            return carry

        lax.fori_loop(0, _CHUNK // _L, blk_body, 0)

    issue(0)
    for c in range(_NCHUNK):
        if c + 1 < _NCHUNK:
            issue(c + 1)
        drain(c)
        compute_chunk(c)

    pltpu.sync_copy(out_v, out.at[pl.ds(base, _BPW)])


@functools.partial(
    pl.kernel,
    out_type=jax.ShapeDtypeStruct((_BATCH,), jnp.float32),
    mesh=plsc.VectorSubcoreMesh(core_axis_name="c", subcore_axis_name="s"),
    scratch_types=[
        pltpu.VMEM((_NCHUNK, _CHUNK), jnp.int32),
        pltpu.VMEM((_NCHUNK, _CHUNK), jnp.int32),
        pltpu.VMEM((_NCHUNK, _CHUNK), jnp.int32),
        pltpu.VMEM((_NCHUNK, _CHUNK), jnp.int32),
        pltpu.VMEM((_NCHUNK, _CHUNK), jnp.int32),
        pltpu.VMEM((_NCHUNK, _CHUNK), jnp.int32),
        pltpu.VMEM((2, _CHUNK, _PW), jnp.float32),
        pltpu.VMEM((2, _CHUNK, _PW), jnp.float32),
        pltpu.VMEM((2, _CHUNK, _PW), jnp.float32),
        pltpu.VMEM((_BPW,), jnp.float32),
        pltpu.SemaphoreType.DMA,
    ],
)
def _dm_sc(e1i, reli, e2i, ent, rel, out, *scratch):
    _dm_body(e1i, reli, e2i, ent, rel, out, *scratch)


def kernel(e1_idx, rel_idx, e2_idx, entity_embedding, rel_embedding):
    e1i = e1_idx.astype(jnp.int32).reshape(_NW * _NCHUNK, _CHUNK)
    reli = rel_idx.astype(jnp.int32).reshape(_NW * _NCHUNK, _CHUNK)
    e2i = e2_idx.astype(jnp.int32).reshape(_NW * _NCHUNK, _CHUNK)
    ent2 = entity_embedding.reshape(_NUM_ENTITIES // 2, _PW)
    rel2 = rel_embedding.reshape(_NUM_RELATIONS // 2, _PW)
    out = _dm_sc(e1i, reli, e2i, ent2, rel2)
    return (out, 0.0)


# speedup vs baseline: 1.6637x; 1.6637x over previous
import functools

import jax
import jax.numpy as jnp
from jax import lax
from jax.experimental import pallas as pl
from jax.experimental.pallas import tpu as pltpu
from jax.experimental.pallas import tpu_sc as plsc

_NUM_ENTITIES = 1000000
_NUM_RELATIONS = 1000
_EMBED_DIM = 64
_BATCH = 16384

_NC = 2
_NS = 16
_L = 16
_NW = _NC * _NS
_BPW = _BATCH // _NW
_HALF = _BPW // 2
_NG = _HALF // _L
_DBLK = _EMBED_DIM // _L

_GDN = lax.GatherDimensionNumbers(
    offset_dims=(), collapsed_slice_dims=(0,), start_index_map=(0,))


def _permute(v, idx):
    return lax.gather(v, idx[:, None], _GDN, slice_sizes=(1,),
                      mode=lax.GatherScatterMode.PROMISE_IN_BOUNDS)


def _hsum_tree(vs, lanes):
    for d in (1, 2, 4, 8):
        perm = lanes ^ d
        mask = (lanes & d) != 0
        nxt = []
        for i in range(0, len(vs), 2):
            a = vs[i] + _permute(vs[i], perm)
            b = vs[i + 1] + _permute(vs[i + 1], perm)
            nxt.append(jnp.where(mask, b, a))
        vs = nxt
    return vs[0]


def _dm_body(e1i, reli, e2i, ent, rel, out,
             idx_e1, idx_r, idx_e2, rows_e1, rows_e2, rel_stage,
             rel_sh, relbuf, out_v, sem, sem_r):
    w = lax.axis_index("s") * _NC + lax.axis_index("c")
    s_id = lax.axis_index("s")
    base = w * _BPW

    pltpu.sync_copy(e1i.at[pl.ds(base, _BPW)], idx_e1)
    pltpu.sync_copy(reli.at[pl.ds(base, _BPW)], idx_r)
    pltpu.sync_copy(e2i.at[pl.ds(base, _BPW)], idx_e2)

    _NRT = _NUM_RELATIONS // 8
    start = jnp.minimum(s_id * 8, _NRT - 8)
    pltpu.sync_copy(rel.at[pl.ds(start, 8)], rel_stage)
    pltpu.sync_copy(rel_stage, rel_sh.at[pl.ds(start, 8)])
    plsc.subcore_barrier()

    def issue(h, g):
        gb = h * _HALF + g * _L
        iv1 = idx_e1[pl.ds(gb, _L)]
        iv2 = idx_e2[pl.ds(gb, _L)]
        for t in range(_L):
            kk = g * (_L // 2) + (t // 2)
            half = pl.ds((t % 2) * _EMBED_DIM, _EMBED_DIM)
            pltpu.async_copy(ent.at[iv1[t]], rows_e1.at[kk, half], sem)
            pltpu.async_copy(ent.at[iv2[t]], rows_e2.at[kk, half], sem)

    def drain(g):
        for t in range(_L):
            kk = g * (_L // 2) + (t // 2)
            half = pl.ds((t % 2) * _EMBED_DIM, _EMBED_DIM)
            pltpu.make_async_copy(ent.at[0], rows_e1.at[kk, half], sem).wait()
            pltpu.make_async_copy(ent.at[0], rows_e2.at[kk, half], sem).wait()

    lanes = lax.iota(jnp.int32, _L)

    def rel_issue(h, g):
        bb = h * _HALF + g * _L
        ivr = idx_r[pl.ds(bb, _L)]
        rt = ivr >> 3
        rs = ivr & 7
        b = g % 2
        for i in range(_L):
            pltpu.async_copy(rel_sh.at[rt[i], rs[i]], relbuf.at[b, i], sem_r)

    def rel_drain(g):
        b = g % 2
        for i in range(_L):
            pltpu.make_async_copy(rel_sh.at[0, 0], relbuf.at[b, i], sem_r).wait()

    def compute_blk(h, g, carry):
        bb = h * _HALF + g * _L
        b = g % 2
        partials = []
        for i in range(_L):
            r2 = g * (_L // 2) + (i // 2)
            off = (i % 2) * _EMBED_DIM
            sl = pl.ds(off, _L)
            acc = rows_e1[r2, sl] * relbuf[b, i, pl.ds(0, _L)] * rows_e2[r2, sl]
            for k in range(1, _DBLK):
                sl = pl.ds(off + k * _L, _L)
                acc = acc + (rows_e1[r2, sl] * relbuf[b, i, pl.ds(k * _L, _L)]
                             * rows_e2[r2, sl])
            partials.append(acc)
        tot = _hsum_tree(partials, lanes)
        y = 1.0 / (1.0 + jnp.exp(-tot))
        out_v[pl.ds(bb, _L)] = y
        return carry

    issue(0, 0)
    issue(0, 1)

    def fetch0(g, carry):
        issue(0, g)
        drain(g - 2)
        return carry

    lax.fori_loop(2, _NG, fetch0, 0)
    drain(_NG - 2)
    drain(_NG - 1)

    rel_issue(0, 0)

    def overlap(g, carry):
        issue(1, g)

        @pl.when(g + 1 < _NG)
        def _():
            rel_issue(0, g + 1)

        rel_drain(g)
        return compute_blk(0, g, carry)

    lax.fori_loop(0, _NG, overlap, 0)

    rel_issue(1, 0)

    def tail(g, carry):
        drain(g)

        @pl.when(g + 1 < _NG)
        def _():
            rel_issue(1, g + 1)

        rel_drain(g)
        return compute_blk(1, g, carry)

    lax.fori_loop(0, _NG, tail, 0)

    pltpu.sync_copy(out_v, out.at[pl.ds(base, _BPW)])


@functools.partial(
    pl.kernel,
    out_type=jax.ShapeDtypeStruct((_BATCH,), jnp.float32),
    mesh=plsc.VectorSubcoreMesh(core_axis_name="c", subcore_axis_name="s"),
    compiler_params=pltpu.CompilerParams(use_tc_tiling_on_sc=True),
    scratch_types=[
        pltpu.VMEM((_BPW,), jnp.int32),
        pltpu.VMEM((_BPW,), jnp.int32),
        pltpu.VMEM((_BPW,), jnp.int32),
        pltpu.VMEM((_HALF // 2, 2 * _EMBED_DIM), jnp.float32),
        pltpu.VMEM((_HALF // 2, 2 * _EMBED_DIM), jnp.float32),
        pltpu.VMEM((8, 8, _EMBED_DIM), jnp.float32),
        pltpu.VMEM_SHARED((_NUM_RELATIONS // 8, 8, _EMBED_DIM), jnp.float32),
        pltpu.VMEM((2, _L, _EMBED_DIM), jnp.float32),
        pltpu.VMEM((_BPW,), jnp.float32),
        pltpu.SemaphoreType.DMA,
        pltpu.SemaphoreType.DMA,
    ],
)
def _dm_sc(e1i, reli, e2i, ent, rel, out, *scratch):
    _dm_body(e1i, reli, e2i, ent, rel, out, *scratch)


def kernel(e1_idx, rel_idx, e2_idx, entity_embedding, rel_embedding):
    e1i = e1_idx.astype(jnp.int32)
    reli = rel_idx.astype(jnp.int32)
    e2i = e2_idx.astype(jnp.int32)
    rel3 = rel_embedding.reshape(_NUM_RELATIONS // 8, 8, _EMBED_DIM)
    out = _dm_sc(e1i, reli, e2i, entity_embedding, rel3)
    return (out, 0.0)
